# baseline (speedup 1.0000x reference)
import jax
import jax.numpy as jnp
from jax.experimental import pallas as pl
from jax.experimental.pallas import tpu as pltpu

_HW = 196
_C = 384
_NCLS = 10
_SHOT = 5
_NEIGHBOR_K = 3


def _top3_colsum(a):
    m1 = jnp.max(a, axis=0)
    eq1 = a == m1[None, :]
    c1 = jnp.sum(jnp.where(eq1, 1.0, 0.0), axis=0)
    a = jnp.where(eq1, -jnp.inf, a)
    m2 = jnp.max(a, axis=0)
    eq2 = a == m2[None, :]
    c2 = jnp.sum(jnp.where(eq2, 1.0, 0.0), axis=0)
    a = jnp.where(eq2, -jnp.inf, a)
    m3 = jnp.max(a, axis=0)
    k1 = jnp.minimum(c1, 3.0)
    k2 = jnp.minimum(c2, 3.0 - k1)
    k3 = 3.0 - k1 - k2
    m2 = jnp.where(k2 > 0, m2, 0.0)
    m3 = jnp.where(k3 > 0, m3, 0.0)
    return jnp.sum(m1 * k1 + m2 * k2 + m3 * k3)


def _knn_body(q_ref, s_ref, o_ref):
    qt = q_ref[0].T
    rq = jax.lax.rsqrt(jnp.sum(qt * qt, axis=1, keepdims=True))
    qn = qt * rq
    total = jnp.float32(0.0)
    for j in range(_SHOT):
        sj = s_ref[0, j]
        ssj = jnp.sum(sj * sj, axis=0)
        rsj = jnp.where(ssj > 0, jax.lax.rsqrt(ssj), 0.0)
        a = jax.lax.dot_general(
            qn, sj, (((1,), (0,)), ((), ())),
            preferred_element_type=jnp.float32,
        )
        a = a * rsj[None, :]
        total = total + _top3_colsum(a)
    o_ref[pl.program_id(0), pl.program_id(1)] = total


def kernel(q, S, qAV_num, SAV_num, shot_num):
    B = q.shape[0]
    q2 = q.reshape(B, _C, _HW)
    s2 = S.reshape(_NCLS, _SHOT, _C, _HW)

    out = pl.pallas_call(
        _knn_body,
        grid=(_NCLS, B),
        in_specs=[
            pl.BlockSpec((1, _C, _HW), lambda c, b: (b, 0, 0)),
            pl.BlockSpec((1, _SHOT, _C, _HW), lambda c, b: (c, 0, 0, 0)),
        ],
        out_specs=pl.BlockSpec(memory_space=pltpu.SMEM),
        out_shape=jax.ShapeDtypeStruct((_NCLS, B), jnp.float32),
    )(q2, s2)
    return out.T


# speedup vs baseline: 1.3496x; 1.3496x over previous
import jax
import jax.numpy as jnp
from jax.experimental import pallas as pl
from jax.experimental.pallas import tpu as pltpu

_HW = 196
_C = 384
_NCLS = 10
_SHOT = 5
_MPAD = 1024
_NEIGHBOR_K = 3


def _top3_colsum(a):
    m1 = jnp.max(a, axis=0)
    eq1 = a == m1[None, :]
    c1 = jnp.sum(jnp.where(eq1, 1.0, 0.0), axis=0)
    a = jnp.where(eq1, -jnp.inf, a)
    m2 = jnp.max(a, axis=0)
    eq2 = a == m2[None, :]
    c2 = jnp.sum(jnp.where(eq2, 1.0, 0.0), axis=0)
    a = jnp.where(eq2, -jnp.inf, a)
    m3 = jnp.max(a, axis=0)
    k1 = jnp.minimum(c1, 3.0)
    k2 = jnp.minimum(c2, 3.0 - k1)
    k3 = 3.0 - k1 - k2
    m2 = jnp.where(k2 > 0, m2, 0.0)
    m3 = jnp.where(k3 > 0, m3, 0.0)
    return jnp.sum(m1 * k1 + m2 * k2 + m3 * k3)


def _knn_body(q_ref, s_ref, o_ref, qn_ref, sn_ref):
    c = pl.program_id(0)
    b = pl.program_id(1)

    @pl.when(c == 0)
    def _stage_query():
        qt = q_ref[0].T
        rq = jax.lax.rsqrt(jnp.sum(qt * qt, axis=1, keepdims=True))
        qn_ref[b] = qt * rq

    @pl.when(b == 0)
    def _stage_support():
        for j in range(_SHOT):
            sj = s_ref[0, j]
            ssj = jnp.sum(sj * sj, axis=0)
            rsj = jnp.where(ssj > 0, jax.lax.rsqrt(ssj), 0.0)
            sn_ref[:, j * _HW:(j + 1) * _HW] = sj * rsj[None, :]
        sn_ref[:, _SHOT * _HW:] = jnp.zeros(
            (_C, _MPAD - _SHOT * _HW), jnp.float32)

    a = jnp.dot(qn_ref[b], sn_ref[...], preferred_element_type=jnp.float32)
    o_ref[c, b] = _top3_colsum(a)


def kernel(q, S, qAV_num, SAV_num, shot_num):
    B = q.shape[0]
    q2 = q.reshape(B, _C, _HW)
    s2 = S.reshape(_NCLS, _SHOT, _C, _HW)

    out = pl.pallas_call(
        _knn_body,
        grid=(_NCLS, B),
        in_specs=[
            pl.BlockSpec((1, _C, _HW), lambda c, b: (b, 0, 0)),
            pl.BlockSpec((1, _SHOT, _C, _HW), lambda c, b: (c, 0, 0, 0)),
        ],
        out_specs=pl.BlockSpec(memory_space=pltpu.SMEM),
        out_shape=jax.ShapeDtypeStruct((_NCLS, B), jnp.float32),
        scratch_shapes=[
            pltpu.VMEM((B, _HW, _C), jnp.float32),
            pltpu.VMEM((_C, _MPAD), jnp.float32),
        ],
    )(q2, s2)
    return out.T


# speedup vs baseline: 1.3904x; 1.0302x over previous
import jax
import jax.numpy as jnp
from jax.experimental import pallas as pl
from jax.experimental.pallas import tpu as pltpu

_HW = 196
_C = 384
_NCLS = 10
_SHOT = 5
_MPAD = 1024
_NEIGHBOR_K = 3


def _top3_colsum(a):
    one = jnp.bfloat16(1.0)
    zero = jnp.bfloat16(0.0)
    m1 = jnp.max(a, axis=0)
    eq1 = a == m1[None, :]
    c1 = jnp.sum(jnp.where(eq1, one, zero), axis=0).astype(jnp.float32)
    a = jnp.where(eq1, -jnp.inf, a)
    m2 = jnp.max(a, axis=0)
    eq2 = a == m2[None, :]
    c2 = jnp.sum(jnp.where(eq2, one, zero), axis=0).astype(jnp.float32)
    a = jnp.where(eq2, -jnp.inf, a)
    m3 = jnp.max(a, axis=0)
    m1 = m1.astype(jnp.float32)
    m2 = m2.astype(jnp.float32)
    m3 = m3.astype(jnp.float32)
    k1 = jnp.minimum(c1, 3.0)
    k2 = jnp.minimum(c2, 3.0 - k1)
    k3 = 3.0 - k1 - k2
    m2 = jnp.where(k2 > 0, m2, 0.0)
    m3 = jnp.where(k3 > 0, m3, 0.0)
    return jnp.sum(m1 * k1 + m2 * k2 + m3 * k3)


def _knn_body(q_ref, s_ref, o_ref, qn_ref, sn_ref):
    c = pl.program_id(0)
    b = pl.program_id(1)

    @pl.when(c == 0)
    def _stage_query():
        qt = q_ref[0].T
        rq = jax.lax.rsqrt(jnp.sum(qt * qt, axis=1, keepdims=True))
        qn_ref[b] = (qt * rq).astype(jnp.bfloat16)

    @pl.when(b == 0)
    def _stage_support():
        for j in range(_SHOT):
            sj = s_ref[0, j]
            ssj = jnp.sum(sj * sj, axis=0)
            rsj = jnp.where(ssj > 0, jax.lax.rsqrt(ssj), 0.0)
            sn_ref[:, j * _HW:(j + 1) * _HW] = (
                sj * rsj[None, :]).astype(jnp.bfloat16)
        sn_ref[:, _SHOT * _HW:] = jnp.zeros(
            (_C, _MPAD - _SHOT * _HW), jnp.bfloat16)

    a = jnp.dot(
        qn_ref[b], sn_ref[...], preferred_element_type=jnp.float32
    ).astype(jnp.bfloat16)
    o_ref[c, b] = _top3_colsum(a)


def kernel(q, S, qAV_num, SAV_num, shot_num):
    B = q.shape[0]
    q2 = q.reshape(B, _C, _HW)
    s2 = S.reshape(_NCLS, _SHOT, _C, _HW)

    out = pl.pallas_call(
        _knn_body,
        grid=(_NCLS, B),
        in_specs=[
            pl.BlockSpec((1, _C, _HW), lambda c, b: (b, 0, 0)),
            pl.BlockSpec((1, _SHOT, _C, _HW), lambda c, b: (c, 0, 0, 0)),
        ],
        out_specs=pl.BlockSpec(memory_space=pltpu.SMEM),
        out_shape=jax.ShapeDtypeStruct((_NCLS, B), jnp.float32),
        scratch_shapes=[
            pltpu.VMEM((B, _HW, _C), jnp.bfloat16),
            pltpu.VMEM((_C, _MPAD), jnp.bfloat16),
        ],
    )(q2, s2)
    return out.T


# speedup vs baseline: 1.4992x; 1.0783x over previous
import jax
import jax.numpy as jnp
from jax.experimental import pallas as pl
from jax.experimental.pallas import tpu as pltpu

_HW = 196
_C = 384
_NCLS = 10
_SHOT = 5
_MPAD = 1024
_NEIGHBOR_K = 3


def _top3_colsum(a):
    chunk = 16
    nfull = (_HW // chunk) * chunk
    neg = -jnp.inf
    m1 = a[0:chunk]
    m2 = jnp.full((chunk, a.shape[1]), neg, jnp.bfloat16)
    m3 = m2
    for i in range(chunk, nfull, chunk):
        r = a[i:i + chunk]
        t1 = jnp.maximum(m1, r)
        b1 = jnp.minimum(m1, r)
        t2 = jnp.maximum(m2, b1)
        b2 = jnp.minimum(m2, b1)
        t3 = jnp.maximum(m3, b2)
        m1, m2, m3 = t1, t2, t3
    a = jnp.concatenate([m1, m2, m3, a[nfull:_HW]], axis=0)
    return _top3_colsum_exact(a)


def _top3_colsum_exact(a):
    one = jnp.bfloat16(1.0)
    zero = jnp.bfloat16(0.0)
    m1 = jnp.max(a, axis=0)
    eq1 = a == m1[None, :]
    c1 = jnp.sum(jnp.where(eq1, one, zero), axis=0).astype(jnp.float32)
    a = jnp.where(eq1, -jnp.inf, a)
    m2 = jnp.max(a, axis=0)
    eq2 = a == m2[None, :]
    c2 = jnp.sum(jnp.where(eq2, one, zero), axis=0).astype(jnp.float32)
    a = jnp.where(eq2, -jnp.inf, a)
    m3 = jnp.max(a, axis=0)
    m1 = m1.astype(jnp.float32)
    m2 = m2.astype(jnp.float32)
    m3 = m3.astype(jnp.float32)
    k1 = jnp.minimum(c1, 3.0)
    k2 = jnp.minimum(c2, 3.0 - k1)
    k3 = 3.0 - k1 - k2
    m2 = jnp.where(k2 > 0, m2, 0.0)
    m3 = jnp.where(k3 > 0, m3, 0.0)
    return jnp.sum(m1 * k1 + m2 * k2 + m3 * k3)


def _knn_body(q_ref, s_ref, o_ref, qn_ref, sn_ref):
    c = pl.program_id(0)
    b = pl.program_id(1)

    @pl.when(c == 0)
    def _stage_query():
        qt = q_ref[0].T
        rq = jax.lax.rsqrt(jnp.sum(qt * qt, axis=1, keepdims=True))
        qn_ref[b] = (qt * rq).astype(jnp.bfloat16)

    @pl.when(b == 0)
    def _stage_support():
        for j in range(_SHOT):
            sj = s_ref[0, j]
            ssj = jnp.sum(sj * sj, axis=0)
            rsj = jnp.where(ssj > 0, jax.lax.rsqrt(ssj), 0.0)
            sn_ref[:, j * _HW:(j + 1) * _HW] = (
                sj * rsj[None, :]).astype(jnp.bfloat16)
        sn_ref[:, _SHOT * _HW:] = jnp.zeros(
            (_C, _MPAD - _SHOT * _HW), jnp.bfloat16)

    a = jnp.dot(
        qn_ref[b], sn_ref[...], preferred_element_type=jnp.float32
    ).astype(jnp.bfloat16)
    o_ref[c, b] = _top3_colsum(a)


def kernel(q, S, qAV_num, SAV_num, shot_num):
    B = q.shape[0]
    q2 = q.reshape(B, _C, _HW)
    s2 = S.reshape(_NCLS, _SHOT, _C, _HW)

    out = pl.pallas_call(
        _knn_body,
        grid=(_NCLS, B),
        in_specs=[
            pl.BlockSpec((1, _C, _HW), lambda c, b: (b, 0, 0)),
            pl.BlockSpec((1, _SHOT, _C, _HW), lambda c, b: (c, 0, 0, 0)),
        ],
        out_specs=pl.BlockSpec(memory_space=pltpu.SMEM),
        out_shape=jax.ShapeDtypeStruct((_NCLS, B), jnp.float32),
        scratch_shapes=[
            pltpu.VMEM((B, _HW, _C), jnp.bfloat16),
            pltpu.VMEM((_C, _MPAD), jnp.bfloat16),
        ],
    )(q2, s2)
    return out.T
